# Initial kernel scaffold; baseline (speedup 1.0000x reference)
import jax
import jax.numpy as jnp
from jax.experimental import pallas as pl


def kernel(scores, K):
    raise NotImplementedError("write your pallas kernel here")


# speedup vs baseline: 14.9716x; 14.9716x over previous
import jax
import jax.numpy as jnp
from jax.experimental import pallas as pl

_ROWS = 128
_N = 1_000_000
_GROUP = 128
_TILE = 16384
_GPT = _TILE // _GROUP
_NT = (_N + _TILE - 1) // _TILE
_G = _NT * _GPT
_K = 100


def _gmax_kernel(x_ref, o_ref):
    t = pl.program_id(0)
    x = x_ref[...]
    col = t * _TILE + jax.lax.broadcasted_iota(jnp.int32, (_ROWS, _TILE), 1)
    x = jnp.where(col < _N, x, -jnp.inf)
    o_ref[...] = jnp.max(x.reshape(_ROWS, _GPT, _GROUP), axis=-1)


def kernel(scores, K):
    gmax = pl.pallas_call(
        _gmax_kernel,
        grid=(_NT,),
        in_specs=[pl.BlockSpec((_ROWS, _TILE), lambda t: (0, t))],
        out_specs=pl.BlockSpec((_ROWS, _GPT), lambda t: (0, t)),
        out_shape=jax.ShapeDtypeStruct((_ROWS, _G), jnp.float32),
    )(scores)

    _, gids = jax.lax.top_k(gmax, _K)
    gids = jnp.sort(gids, axis=-1)

    cols = gids[:, :, None] * _GROUP + jnp.arange(_GROUP, dtype=gids.dtype)
    cols = cols.reshape(_ROWS, _K * _GROUP)
    valid = cols < _N
    cand = jnp.take_along_axis(scores, jnp.where(valid, cols, 0), axis=1)
    cand = jnp.where(valid, cand, -jnp.inf)

    top_vals, pos = jax.lax.top_k(cand, _K)
    top_inds = jnp.take_along_axis(cols, pos, axis=1).astype(jnp.int32)
    return top_vals, top_inds


# speedup vs baseline: 20.7123x; 1.3834x over previous
import jax
import jax.numpy as jnp
from jax.experimental import pallas as pl

_ROWS = 128
_N = 1_000_000
_GROUP = 128
_TILE = 16384
_GPT = _TILE // _GROUP
_NT = (_N + _TILE - 1) // _TILE
_G = _NT * _GPT
_K = 100


def _gmax_kernel(x_ref, o_ref):
    t = pl.program_id(0)
    x = x_ref[...]
    col = t * _TILE + jax.lax.broadcasted_iota(jnp.int32, (_ROWS, _TILE), 1)
    x = jnp.where(col < _N, x, -jnp.inf)
    o_ref[...] = jnp.max(x.reshape(_ROWS, _GPT, _GROUP), axis=-1)


def _group_select_kernel(g_ref, ids_ref):
    gid = jax.lax.broadcasted_iota(jnp.int32, (_ROWS, _G), 1)
    slot = jax.lax.broadcasted_iota(jnp.int32, (_ROWS, _K), 1)

    def body(i, carry):
        x, ids = carry
        m = jnp.max(x, axis=1, keepdims=True)
        mi = jnp.where(x == m, gid, _G)
        idx = jnp.min(mi, axis=1, keepdims=True)
        ids = jnp.where(slot == i, idx, ids)
        return jnp.where(mi == idx, -jnp.inf, x), ids

    _, ids = jax.lax.fori_loop(
        0, _K, body, (g_ref[...], jnp.zeros((_ROWS, _K), jnp.int32)),
        unroll=False)
    ids_ref[...] = ids


def _final_select_kernel(c_ref, ids_ref, vals_ref, inds_ref):
    lane = jax.lax.broadcasted_iota(jnp.int32, (_ROWS, _K, _GROUP), 2)
    col = ids_ref[...][:, :, None] * _GROUP + lane
    big = jnp.int32(_N)
    x0 = jnp.where(col < _N, c_ref[...], -jnp.inf)
    slot = jax.lax.broadcasted_iota(jnp.int32, (_ROWS, _K), 1)

    def body(i, carry):
        x, vals, inds = carry
        m = jnp.max(x, axis=(1, 2), keepdims=True)
        mi = jnp.where(x == m, col, big)
        idx = jnp.min(mi, axis=(1, 2), keepdims=True)
        vals = jnp.where(slot == i, m.reshape(_ROWS, 1), vals)
        inds = jnp.where(slot == i, idx.reshape(_ROWS, 1), inds)
        return jnp.where(mi == idx, -jnp.inf, x), vals, inds

    _, vals, inds = jax.lax.fori_loop(
        0, _K, body,
        (x0, jnp.zeros((_ROWS, _K), jnp.float32),
         jnp.zeros((_ROWS, _K), jnp.int32)),
        unroll=False)
    vals_ref[...] = vals
    inds_ref[...] = inds


def kernel(scores, K):
    gmax = pl.pallas_call(
        _gmax_kernel,
        grid=(_NT,),
        in_specs=[pl.BlockSpec((_ROWS, _TILE), lambda t: (0, t))],
        out_specs=pl.BlockSpec((_ROWS, _GPT), lambda t: (0, t)),
        out_shape=jax.ShapeDtypeStruct((_ROWS, _G), jnp.float32),
    )(scores)

    gids = pl.pallas_call(
        _group_select_kernel,
        out_shape=jax.ShapeDtypeStruct((_ROWS, _K), jnp.int32),
    )(gmax)

    cols = gids[:, :, None] * _GROUP + jnp.arange(_GROUP, dtype=jnp.int32)
    cols = cols.reshape(_ROWS, _K * _GROUP)
    cand = jnp.take_along_axis(scores, jnp.minimum(cols, _N - 1), axis=1)
    cand = cand.reshape(_ROWS, _K, _GROUP)

    top_vals, top_inds = pl.pallas_call(
        _final_select_kernel,
        out_shape=(
            jax.ShapeDtypeStruct((_ROWS, _K), jnp.float32),
            jax.ShapeDtypeStruct((_ROWS, _K), jnp.int32),
        ),
    )(cand, gids)
    return top_vals, top_inds
